# baseline (speedup 1.0000x reference)
import functools

import jax
import jax.numpy as jnp
from jax.experimental import pallas as pl
from jax.experimental.pallas import tpu as pltpu

_C_HI, _C_LO = 14, 4
_LANE = 128


def _reduce_body(*refs, nblk, n_streams, c_hi, c_lo):
    x_refs = refs[:n_streams]
    msum_ref, csum_ref = refs[n_streams], refs[n_streams + 1]
    hb = pl.program_id(0)
    b = pl.program_id(1)

    @pl.when(b == 0)
    def _init():
        msum_ref[...] = jnp.zeros_like(msum_ref)
        csum_ref[...] = jnp.zeros_like(csum_ref)

    total = None
    parts = []
    for x_ref in x_refs:
        x = x_ref[0]
        rpl, vp = x.shape
        p = x.reshape(rpl // 8, 8, vp).astype(jnp.float32).sum(axis=0)
        parts.append(p)
        total = p if total is None else total + p
    msum_ref[0] += total

    step = (hb * nblk + b) * n_streams
    for s, p in enumerate(parts):
        @pl.when(step + s == c_hi)
        def _hi(p=p):
            csum_ref[0] += p

        @pl.when(step + s == c_lo)
        def _lo(p=p):
            csum_ref[0] -= p


def _attn_reduce(image_attn, c_hi, c_lo, n_streams=2):
    L, H, Q, Vp = image_attn.shape
    assert Vp % _LANE == 0
    assert L % 2 == 0, "megacore split over layer halves needs even L"
    assert (L // 2) % n_streams == 0

    rows_per_layer = H * Q
    rows = L * rows_per_layer
    nblk = (L // 2) // n_streams

    flat = image_attn.reshape(L, rows_per_layer, Vp)

    body = functools.partial(
        _reduce_body, nblk=nblk, n_streams=n_streams, c_hi=c_hi, c_lo=c_lo)

    def mk_index(s):
        return lambda hb, b: ((hb * nblk + b) * n_streams + s, 0, 0)

    msum, csum = pl.pallas_call(
        body,
        out_shape=(
            jax.ShapeDtypeStruct((2, 8, Vp), jnp.float32),
            jax.ShapeDtypeStruct((2, 8, Vp), jnp.float32),
        ),
        grid=(2, nblk),
        in_specs=[pl.BlockSpec((1, rows_per_layer, Vp), mk_index(s))
                  for s in range(n_streams)],
        out_specs=(
            pl.BlockSpec((1, 8, Vp), lambda hb, b: (hb, 0, 0)),
            pl.BlockSpec((1, 8, Vp), lambda hb, b: (hb, 0, 0)),
        ),
        compiler_params=pltpu.CompilerParams(
            dimension_semantics=("parallel", "arbitrary")),
    )(*([flat] * n_streams))

    mean_attn = jnp.sum(msum, axis=(0, 1)) / float(rows)
    contr = jnp.sum(csum, axis=(0, 1)) / float(rows_per_layer)
    return mean_attn, jnp.maximum(contr, 0.0)


def kernel(image_attn):
    return _attn_reduce(image_attn, _C_HI, _C_LO, n_streams=2)
